# baseline (speedup 1.0000x reference)
import functools

import jax
import jax.numpy as jnp
from jax import lax
from jax.experimental import pallas as pl
from jax.experimental.pallas import tpu as pltpu
from jax.experimental.pallas import tpu_sc as plsc

B = 16384
D = 64
H1 = 128
NU = 1000000
NI = 100000
GU = NU // 2
GI = NI // 2
NC, NS = 2, 16
NW = NC * NS
BPW = B // NW
P = 128


def _bf16_bits(x):
    u = lax.bitcast_convert_type(x, jnp.int32)
    r = u + 0x7FFF + ((lax.shift_right_logical(u, 16)) & 1)
    return lax.shift_right_logical(r, 16)


def _product_body(tt_ref, a1p_ref, out_ref):
    h = lax.dot_general(tt_ref[...], a1p_ref[...], (((0,), (0,)), ((), ())),
                        preferred_element_type=jnp.float32)
    hr = h.reshape(h.shape[0] // 2, 2, H1)
    re = _bf16_bits(hr[:, 0, :])
    ro = _bf16_bits(hr[:, 1, :])
    out_ref[...] = re | lax.shift_left(ro, 16)


def _product(tt, a1p, n_entries, bn=2048):
    grid = ((n_entries + bn - 1) // bn,)
    return pl.pallas_call(
        _product_body,
        out_shape=jax.ShapeDtypeStruct((n_entries // 2, H1), jnp.int32),
        grid=grid,
        in_specs=[
            pl.BlockSpec((D, bn), lambda m: (0, m)),
            pl.BlockSpec((D, H1), lambda m: (0, 0)),
        ],
        out_specs=pl.BlockSpec((bn // 2, H1), lambda m: (m, 0)),
    )(tt, a1p)


@functools.cache
def _make_sc_gather():
    mesh = plsc.VectorSubcoreMesh(
        core_axis_name="c", subcore_axis_name="s", num_cores=NC, num_subcores=NS)

    @functools.partial(
        pl.kernel,
        out_type=(
            jax.ShapeDtypeStruct((B, H1), jnp.int32),
            jax.ShapeDtypeStruct((B, H1), jnp.int32),
        ),
        mesh=mesh,
        scratch_types=[
            pltpu.VMEM((BPW,), jnp.int32),
            pltpu.VMEM((BPW,), jnp.int32),
            pltpu.VMEM((P, H1), jnp.int32),
            pltpu.VMEM((P, H1), jnp.int32),
            pltpu.SemaphoreType.DMA,
        ],
    )
    def _sc_gather(gu_hbm, gi_hbm, pu_hbm, pi_hbm, ou_hbm, oi_hbm,
                   gu_v, gi_v, us_v, is_v, sem):
        wid = lax.axis_index("s") * NC + lax.axis_index("c")
        base = wid * BPW
        pltpu.sync_copy(gu_hbm.at[pl.ds(base, BPW)], gu_v)
        pltpu.sync_copy(gi_hbm.at[pl.ds(base, BPW)], gi_v)
        for p in range(BPW // P):
            col = base + p * P
            pltpu.make_async_copy(
                pu_hbm.at[gu_v.at[pl.ds(p * P, P)]], us_v, sem).start()
            pltpu.make_async_copy(
                pi_hbm.at[gi_v.at[pl.ds(p * P, P)]], is_v, sem).start()
            pltpu.make_async_copy(pu_hbm.at[pl.ds(0, P)], us_v, sem).wait()
            pltpu.make_async_copy(pi_hbm.at[pl.ds(0, P)], is_v, sem).wait()
            pltpu.sync_copy(us_v, ou_hbm.at[pl.ds(col, P)])
            pltpu.sync_copy(is_v, oi_hbm.at[pl.ds(col, P)])

    return _sc_gather


def _select_unpack(w, parity):
    half = jnp.where(parity == 1, lax.shift_right_logical(w, 16), w & 0xFFFF)
    return lax.bitcast_convert_type(lax.shift_left(half, 16), jnp.float32)


def _final_body(gu_ref, gi_ref, mu_ref, mi_ref, c1_ref, w23_ref, c3_ref, out_ref):
    xu = _select_unpack(gu_ref[...], mu_ref[...])
    xi = _select_unpack(gi_ref[...], mi_ref[...])
    h = jnp.maximum(xu + xi + c1_ref[...], 0.0)
    out_ref[...] = (
        jnp.dot(h, w23_ref[...], preferred_element_type=jnp.float32) + c3_ref[...])


def _final(gu, gi, mu, mi, c1, w23, c3, bm=2048):
    grid = (B // bm,)
    return pl.pallas_call(
        _final_body,
        out_shape=jax.ShapeDtypeStruct((B, 1), jnp.float32),
        grid=grid,
        in_specs=[
            pl.BlockSpec((bm, H1), lambda m: (m, 0)),
            pl.BlockSpec((bm, H1), lambda m: (m, 0)),
            pl.BlockSpec((bm, 1), lambda m: (m, 0)),
            pl.BlockSpec((bm, 1), lambda m: (m, 0)),
            pl.BlockSpec((1, H1), lambda m: (0, 0)),
            pl.BlockSpec((H1, 1), lambda m: (0, 0)),
            pl.BlockSpec((1, 1), lambda m: (0, 0)),
        ],
        out_specs=pl.BlockSpec((bm, 1), lambda m: (m, 0)),
    )(gu, gi, mu, mi, c1, w23, c3)


def kernel(user, item, user_table, item_table, g0, be0, W1, b1, g1, be1, W2, b2, W3, b3):
    s = 1.0 / jnp.sqrt(1.0 + 1e-5)
    g0p = g0 * s
    g1p = g1 * s
    A1 = W1 * g0p[None, :] * g1p[:, None]
    c1 = g1p * (W1 @ be0 + b1) + be1
    w23 = (W3 @ W2).T
    c3 = (W3 @ b2 + b3).reshape(1, 1)
    a1u = A1[:, :D].T
    a1i = A1[:, D:].T

    utt = user_table.T
    itt = item_table.T

    pu = _product(utt, a1u, NU)
    pi = _product(itt, a1i, NI)

    ui = user.astype(jnp.int32)
    ii = item.astype(jnp.int32)
    gu, gi = _make_sc_gather()(ui // 2, ii // 2, pu, pi)

    mu = (ui % 2).reshape(B, 1)
    mi = (ii % 2).reshape(B, 1)
    out = _final(gu, gi, mu, mi, c1.reshape(1, H1), w23, c3)
    return out.reshape(B)


# speedup vs baseline: 1.7247x; 1.7247x over previous
import functools

import jax
import jax.numpy as jnp
from jax import lax
from jax.experimental import pallas as pl
from jax.experimental.pallas import tpu as pltpu
from jax.experimental.pallas import tpu_sc as plsc

B = 16384
D = 64
H1 = 128
NU = 1000000
NI = 100000
GU = NU // 2
GI = NI // 2
NC, NS = 2, 16
NW = NC * NS
BPW = B // NW
P = 128


def _repack_body(tt_ref, eye_ref, out_ref):
    t = lax.dot_general(tt_ref[...], eye_ref[...], (((0,), (0,)), ((), ())),
                        preferred_element_type=jnp.float32)
    half = t.shape[0] // 2
    out_ref[...] = jnp.concatenate([t[:half, :], t[half:, :]], axis=1)


def _repack(tt, eye, n_entries, bn=2048):
    nb = (n_entries + bn - 1) // bn
    grid = (nb,)
    return pl.pallas_call(
        _repack_body,
        out_shape=jax.ShapeDtypeStruct((nb * (bn // 2), 2 * D), jnp.float32),
        grid=grid,
        in_specs=[
            pl.BlockSpec((D, bn), lambda m: (0, m)),
            pl.BlockSpec((D, D), lambda m: (0, 0)),
        ],
        out_specs=pl.BlockSpec((bn // 2, 2 * D), lambda m: (m, 0)),
    )(tt, eye)


@functools.cache
def _make_sc_gather():
    mesh = plsc.VectorSubcoreMesh(
        core_axis_name="c", subcore_axis_name="s", num_cores=NC, num_subcores=NS)

    @functools.partial(
        pl.kernel,
        out_type=(
            jax.ShapeDtypeStruct((B, 2 * D), jnp.float32),
            jax.ShapeDtypeStruct((B, 2 * D), jnp.float32),
        ),
        mesh=mesh,
        scratch_types=[
            pltpu.VMEM((BPW,), jnp.int32),
            pltpu.VMEM((BPW,), jnp.int32),
            pltpu.VMEM((P, 2 * D), jnp.float32),
            pltpu.VMEM((P, 2 * D), jnp.float32),
            pltpu.SemaphoreType.DMA,
        ],
    )
    def _sc_gather(gu_hbm, gi_hbm, pu_hbm, pi_hbm, ou_hbm, oi_hbm,
                   gu_v, gi_v, us_v, is_v, sem):
        wid = lax.axis_index("s") * NC + lax.axis_index("c")
        base = wid * BPW
        pltpu.sync_copy(gu_hbm.at[pl.ds(base, BPW)], gu_v)
        pltpu.sync_copy(gi_hbm.at[pl.ds(base, BPW)], gi_v)
        for p in range(BPW // P):
            col = base + p * P
            pltpu.make_async_copy(
                pu_hbm.at[gu_v.at[pl.ds(p * P, P)]], us_v, sem).start()
            pltpu.make_async_copy(
                pi_hbm.at[gi_v.at[pl.ds(p * P, P)]], is_v, sem).start()
            pltpu.make_async_copy(pu_hbm.at[pl.ds(0, P)], us_v, sem).wait()
            pltpu.make_async_copy(pi_hbm.at[pl.ds(0, P)], is_v, sem).wait()
            pltpu.sync_copy(us_v, ou_hbm.at[pl.ds(col, P)])
            pltpu.sync_copy(is_v, oi_hbm.at[pl.ds(col, P)])

    return _sc_gather


def _final_body(gu_ref, gi_ref, mu_ref, mi_ref, a1u_ref, a1i_ref,
                c1_ref, w23_ref, c3_ref, out_ref):
    mu = (mu_ref[...] == 1)
    mi = (mi_ref[...] == 1)
    xu = jnp.where(mu, gu_ref[:, D:], gu_ref[:, :D])
    xi = jnp.where(mi, gi_ref[:, D:], gi_ref[:, :D])
    h = jnp.dot(xu, a1u_ref[...], preferred_element_type=jnp.float32)
    h = h + jnp.dot(xi, a1i_ref[...], preferred_element_type=jnp.float32)
    h = jnp.maximum(h + c1_ref[...], 0.0)
    out_ref[...] = (
        jnp.dot(h, w23_ref[...], preferred_element_type=jnp.float32) + c3_ref[...])


def _final(gu, gi, mu, mi, a1u, a1i, c1, w23, c3, bm=2048):
    grid = (B // bm,)
    return pl.pallas_call(
        _final_body,
        out_shape=jax.ShapeDtypeStruct((B, 1), jnp.float32),
        grid=grid,
        in_specs=[
            pl.BlockSpec((bm, 2 * D), lambda m: (m, 0)),
            pl.BlockSpec((bm, 2 * D), lambda m: (m, 0)),
            pl.BlockSpec((bm, 1), lambda m: (m, 0)),
            pl.BlockSpec((bm, 1), lambda m: (m, 0)),
            pl.BlockSpec((D, H1), lambda m: (0, 0)),
            pl.BlockSpec((D, H1), lambda m: (0, 0)),
            pl.BlockSpec((1, H1), lambda m: (0, 0)),
            pl.BlockSpec((H1, 1), lambda m: (0, 0)),
            pl.BlockSpec((1, 1), lambda m: (0, 0)),
        ],
        out_specs=pl.BlockSpec((bm, 1), lambda m: (m, 0)),
    )(gu, gi, mu, mi, a1u, a1i, c1, w23, c3)


def kernel(user, item, user_table, item_table, g0, be0, W1, b1, g1, be1, W2, b2, W3, b3):
    s = 1.0 / jnp.sqrt(1.0 + 1e-5)
    g0p = g0 * s
    g1p = g1 * s
    A1 = W1 * g0p[None, :] * g1p[:, None]
    c1 = g1p * (W1 @ be0 + b1) + be1
    w23 = (W3 @ W2).T
    c3 = (W3 @ b2 + b3).reshape(1, 1)
    a1u = A1[:, :D].T
    a1i = A1[:, D:].T

    utt = user_table.T
    itt = item_table.T

    eye = jnp.eye(D, dtype=jnp.float32)
    pu = _repack(utt, eye, NU)
    pi = _repack(itt, eye, NI)

    ui = user.astype(jnp.int32)
    ii = item.astype(jnp.int32)

    bn, hbn = 2048, 1024
    ku = ui % bn
    ki = ii % bn
    gur = (ui // bn) * hbn + jnp.where(ku >= hbn, ku - hbn, ku)
    gir = (ii // bn) * hbn + jnp.where(ki >= hbn, ki - hbn, ki)
    gu, gi = _make_sc_gather()(gur, gir, pu, pi)

    mu = (ku >= hbn).astype(jnp.int32).reshape(B, 1)
    mi = (ki >= hbn).astype(jnp.int32).reshape(B, 1)
    out = _final(gu, gi, mu, mi, a1u, a1i, c1.reshape(1, H1), w23, c3)
    return out.reshape(B)


# speedup vs baseline: 2.3328x; 1.3526x over previous
import functools

import jax
import jax.numpy as jnp
from jax import lax
from jax.experimental import pallas as pl
from jax.experimental.pallas import tpu as pltpu
from jax.experimental.pallas import tpu_sc as plsc

B = 16384
D = 64
H1 = 128
NC, NS = 2, 16
NW = NC * NS
BPW = B // NW


@functools.cache
def _make_sc_gather():
    mesh = plsc.VectorSubcoreMesh(
        core_axis_name="c", subcore_axis_name="s", num_cores=NC, num_subcores=NS)

    @functools.partial(
        pl.kernel,
        out_type=(
            jax.ShapeDtypeStruct((B, D), jnp.float32),
            jax.ShapeDtypeStruct((B, D), jnp.float32),
        ),
        mesh=mesh,
        scratch_types=[
            pltpu.VMEM((BPW,), jnp.int32),
            pltpu.VMEM((BPW,), jnp.int32),
            pltpu.VMEM((BPW // 2, D), jnp.float32),
            pltpu.VMEM((BPW // 2, D), jnp.float32),
            pltpu.SemaphoreType.DMA,
        ],
    )
    def _sc_gather(uidx_hbm, iidx_hbm, utab_hbm, itab_hbm, uout_hbm, iout_hbm,
                   uidx_v, iidx_v, urows_v, irows_v, sem):
        wid = lax.axis_index("s") * NC + lax.axis_index("c")
        base = wid * BPW
        pltpu.sync_copy(uidx_hbm.at[pl.ds(base, BPW)], uidx_v)
        pltpu.sync_copy(iidx_hbm.at[pl.ds(base, BPW)], iidx_v)

        half = BPW // 2
        for h in range(2):
            hb = h * half

            def group(g, _):
                gb = g * 16
                vu = uidx_v[pl.ds(hb + gb, 16)]
                vi = iidx_v[pl.ds(hb + gb, 16)]
                for k in range(16):
                    pltpu.make_async_copy(
                        utab_hbm.at[pl.ds(vu[k], 1)],
                        urows_v.at[pl.ds(gb + k, 1)], sem).start()
                    pltpu.make_async_copy(
                        itab_hbm.at[pl.ds(vi[k], 1)],
                        irows_v.at[pl.ds(gb + k, 1)], sem).start()
                return ()

            lax.fori_loop(0, half // 16, group, ())
            pltpu.make_async_copy(utab_hbm.at[pl.ds(0, half)], urows_v, sem).wait()
            pltpu.make_async_copy(itab_hbm.at[pl.ds(0, half)], irows_v, sem).wait()
            pltpu.sync_copy(urows_v, uout_hbm.at[pl.ds(base + hb, half)])
            pltpu.sync_copy(irows_v, iout_hbm.at[pl.ds(base + hb, half)])

    return _sc_gather


def _mlp_body(u_ref, i_ref, a1u_ref, a1i_ref, c1_ref, w23_ref, c3_ref, out_ref):
    h = jnp.dot(u_ref[...], a1u_ref[...], preferred_element_type=jnp.float32)
    h = h + jnp.dot(i_ref[...], a1i_ref[...], preferred_element_type=jnp.float32)
    h = jnp.maximum(h + c1_ref[...], 0.0)
    out_ref[...] = (
        jnp.dot(h, w23_ref[...], preferred_element_type=jnp.float32) + c3_ref[...])


def _mlp(u, i, a1u, a1i, c1, w23, c3, bm=2048):
    grid = (B // bm,)
    return pl.pallas_call(
        _mlp_body,
        out_shape=jax.ShapeDtypeStruct((B, 1), jnp.float32),
        grid=grid,
        in_specs=[
            pl.BlockSpec((bm, D), lambda m: (m, 0)),
            pl.BlockSpec((bm, D), lambda m: (m, 0)),
            pl.BlockSpec((D, H1), lambda m: (0, 0)),
            pl.BlockSpec((D, H1), lambda m: (0, 0)),
            pl.BlockSpec((1, H1), lambda m: (0, 0)),
            pl.BlockSpec((H1, 1), lambda m: (0, 0)),
            pl.BlockSpec((1, 1), lambda m: (0, 0)),
        ],
        out_specs=pl.BlockSpec((bm, 1), lambda m: (m, 0)),
    )(u, i, a1u, a1i, c1, w23, c3)


def kernel(user, item, user_table, item_table, g0, be0, W1, b1, g1, be1, W2, b2, W3, b3):
    s = 1.0 / jnp.sqrt(1.0 + 1e-5)
    g0p = g0 * s
    g1p = g1 * s
    A1 = W1 * g0p[None, :] * g1p[:, None]
    c1 = g1p * (W1 @ be0 + b1) + be1
    w23 = (W3 @ W2).T
    c3 = (W3 @ b2 + b3).reshape(1, 1)
    a1u = A1[:, :D].T
    a1i = A1[:, D:].T

    uidx = user.astype(jnp.int32)
    iidx = item.astype(jnp.int32)
    u_emb, i_emb = _make_sc_gather()(uidx, iidx, user_table, item_table)
    out = _mlp(u_emb, i_emb, a1u, a1i, c1.reshape(1, H1), w23, c3)
    return out.reshape(B)


# speedup vs baseline: 2.3328x; 1.0000x over previous
import functools

import jax
import jax.numpy as jnp
from jax import lax
from jax.experimental import pallas as pl
from jax.experimental.pallas import tpu as pltpu
from jax.experimental.pallas import tpu_sc as plsc

B = 16384
D = 64
H1 = 128
NC, NS = 2, 16
NW = NC * NS
BPW = B // NW


@functools.cache
def _make_sc_gather():
    mesh = plsc.VectorSubcoreMesh(
        core_axis_name="c", subcore_axis_name="s", num_cores=NC, num_subcores=NS)

    @functools.partial(
        pl.kernel,
        out_type=jax.ShapeDtypeStruct((B, D), jnp.float32),
        mesh=mesh,
        scratch_types=[
            pltpu.VMEM((BPW,), jnp.int32),
            pltpu.VMEM((BPW // 2, D), jnp.float32),
            pltpu.SemaphoreType.DMA,
        ],
    )
    def _sc_gather(idx_hbm, tab_hbm, out_hbm, idx_v, rows_v, sem):
        wid = lax.axis_index("s") * NC + lax.axis_index("c")
        base = wid * BPW
        pltpu.sync_copy(idx_hbm.at[pl.ds(base, BPW)], idx_v)

        half = BPW // 2
        for h in range(2):
            hb = h * half

            def group(g, _):
                gb = g * 16
                v = idx_v[pl.ds(hb + gb, 16)]
                for k in range(16):
                    pltpu.make_async_copy(
                        tab_hbm.at[pl.ds(v[k], 1)],
                        rows_v.at[pl.ds(gb + k, 1)], sem).start()
                return ()

            lax.fori_loop(0, half // 16, group, ())
            pltpu.make_async_copy(tab_hbm.at[pl.ds(0, half)], rows_v, sem).wait()
            pltpu.sync_copy(rows_v, out_hbm.at[pl.ds(base + hb, half)])

    return _sc_gather


def _mlp_body(u_ref, i_ref, a1u_ref, a1i_ref, c1_ref, w23_ref, c3_ref, out_ref):
    h = jnp.dot(u_ref[...], a1u_ref[...], preferred_element_type=jnp.float32)
    h = h + jnp.dot(i_ref[...], a1i_ref[...], preferred_element_type=jnp.float32)
    h = jnp.maximum(h + c1_ref[...], 0.0)
    out_ref[...] = (
        jnp.dot(h, w23_ref[...], preferred_element_type=jnp.float32) + c3_ref[...])


def _mlp(u, i, a1u, a1i, c1, w23, c3, bm=2048):
    grid = (B // bm,)
    return pl.pallas_call(
        _mlp_body,
        out_shape=jax.ShapeDtypeStruct((B, 1), jnp.float32),
        grid=grid,
        in_specs=[
            pl.BlockSpec((bm, D), lambda m: (m, 0)),
            pl.BlockSpec((bm, D), lambda m: (m, 0)),
            pl.BlockSpec((D, H1), lambda m: (0, 0)),
            pl.BlockSpec((D, H1), lambda m: (0, 0)),
            pl.BlockSpec((1, H1), lambda m: (0, 0)),
            pl.BlockSpec((H1, 1), lambda m: (0, 0)),
            pl.BlockSpec((1, 1), lambda m: (0, 0)),
        ],
        out_specs=pl.BlockSpec((bm, 1), lambda m: (m, 0)),
    )(u, i, a1u, a1i, c1, w23, c3)


def kernel(user, item, user_table, item_table, g0, be0, W1, b1, g1, be1, W2, b2, W3, b3):
    s = 1.0 / jnp.sqrt(1.0 + 1e-5)
    g0p = g0 * s
    g1p = g1 * s
    A1 = W1 * g0p[None, :] * g1p[:, None]
    c1 = g1p * (W1 @ be0 + b1) + be1
    w23 = (W3 @ W2).T
    c3 = (W3 @ b2 + b3).reshape(1, 1)
    a1u = A1[:, :D].T
    a1i = A1[:, D:].T

    uidx = user.astype(jnp.int32)
    iidx = item.astype(jnp.int32)
    gather = _make_sc_gather()
    i_emb = gather(iidx, item_table)
    u_emb = gather(uidx, user_table)
    out = _mlp(u_emb, i_emb, a1u, a1i, c1.reshape(1, H1), w23, c3)
    return out.reshape(B)
